# Initial kernel scaffold; baseline (speedup 1.0000x reference)
import functools

import jax
import jax.numpy as jnp
from jax import lax
from jax.experimental import pallas as pl
from jax.experimental.pallas import tpu as pltpu
from jax.experimental.pallas import tpu_sc as plsc

_NC = 2
_NS = 16
_LANES = 16


def _sc_aggregate(x, alpha_r, idxi_r, idxj_r, n_nodes, feat, n_chunks, chunk):
    rows_per_tile = n_nodes // _NS
    zrows = rows_per_tile // 5
    mesh = plsc.VectorSubcoreMesh(core_axis_name="c", subcore_axis_name="s")

    @functools.partial(
        pl.kernel,
        out_type=jax.ShapeDtypeStruct((_NC * n_nodes, feat), jnp.float32),
        mesh=mesh,
        scratch_types=[
            pltpu.VMEM((n_chunks, chunk), jnp.int32),
            pltpu.VMEM((n_chunks, chunk), jnp.int32),
            pltpu.VMEM((n_chunks, chunk), jnp.float32),
            pltpu.VMEM((chunk, feat), jnp.float32),
            pltpu.VMEM((zrows, feat), jnp.float32),
            pltpu.VMEM_SHARED((n_nodes, feat), jnp.float32),
            pltpu.SemaphoreType.DMA,
        ],
    )
    def body(x_hbm, alpha_hbm, idxi_hbm, idxj_hbm, out_hbm,
             idxj_v, idxi_v, alpha_v, g, zbuf, acc, sem):
        c = lax.axis_index("c")
        s = lax.axis_index("s")
        w = c * _NS + s

        pltpu.sync_copy(idxj_hbm.at[w], idxj_v)
        pltpu.sync_copy(idxi_hbm.at[w], idxi_v)
        pltpu.sync_copy(alpha_hbm.at[w], alpha_v)

        @pl.loop(0, zrows)
        def _zrow(i):
            for t in range(feat // _LANES):
                zbuf[i, pl.ds(t * _LANES, _LANES)] = jnp.zeros(
                    (_LANES,), jnp.float32)

        base = s * rows_per_tile
        for z in range(5):
            pltpu.sync_copy(zbuf, acc.at[pl.ds(base + z * zrows, zrows)])
        plsc.subcore_barrier()

        @pl.loop(0, n_chunks)
        def _chunk(k):
            pltpu.async_copy(x_hbm.at[idxj_v.at[k]], g, sem).wait()

            @pl.loop(0, chunk)
            def _srow(r):
                a = alpha_v[k, r]
                for t in range(feat // _LANES):
                    sl = pl.ds(t * _LANES, _LANES)
                    g[r, sl] = g[r, sl] * a

            pltpu.sync_copy(g, acc.at[idxi_v.at[k]], add=True)

        plsc.subcore_barrier()
        pltpu.sync_copy(acc.at[pl.ds(base, rows_per_tile)],
                        out_hbm.at[pl.ds(c * n_nodes + base, rows_per_tile)])

    return body(x, alpha_r, idxi_r, idxj_r)


def _tc_finish(p0, p1, W, n_nodes, feat, block):
    def body(p0_ref, p1_ref, w_ref, o_ref):
        o_ref[...] = jnp.dot(p0_ref[...] + p1_ref[...], w_ref[...],
                             preferred_element_type=jnp.float32)

    return pl.pallas_call(
        body,
        grid=(n_nodes // block,),
        in_specs=[
            pl.BlockSpec((block, feat), lambda i: (i, 0)),
            pl.BlockSpec((block, feat), lambda i: (i, 0)),
            pl.BlockSpec((feat, feat), lambda i: (0, 0)),
        ],
        out_specs=pl.BlockSpec((block, feat), lambda i: (i, 0)),
        out_shape=jax.ShapeDtypeStruct((n_nodes, feat), jnp.float32),
    )(p0, p1, W)


def kernel(x, alpha_ij, idx_i, idx_j, W):
    n_nodes, feat = x.shape
    n_edges = alpha_ij.shape[0]
    nw = _NC * _NS
    chunk = 80
    n_chunks = n_edges // (nw * chunk)
    assert n_chunks * nw * chunk == n_edges

    idxi_r = idx_i.astype(jnp.int32).reshape(nw, n_chunks, chunk)
    idxj_r = idx_j.astype(jnp.int32).reshape(nw, n_chunks, chunk)
    alpha_r = alpha_ij.astype(jnp.float32).reshape(nw, n_chunks, chunk)

    partial = _sc_aggregate(x.astype(jnp.float32), alpha_r, idxi_r, idxj_r,
                            n_nodes, feat, n_chunks, chunk)
    return _tc_finish(partial[:n_nodes], partial[n_nodes:],
                      W.astype(jnp.float32), n_nodes, feat, 500)


# speedup vs baseline: 6.2039x; 6.2039x over previous
import functools

import jax
import jax.numpy as jnp
from jax import lax
from jax.experimental import pallas as pl
from jax.experimental.pallas import tpu as pltpu
from jax.experimental.pallas import tpu_sc as plsc

_NC = 2
_NS = 16
_LANES = 16


def _sc_aggregate(x, alpha_r, idxi_r, idxj_r, n_nodes, feat,
                  n_super, chunks_per_super, chunk):
    rows_per_tile = n_nodes // _NS
    n_z = 25
    zrows = rows_per_tile // n_z
    mesh = plsc.VectorSubcoreMesh(core_axis_name="c", subcore_axis_name="s")

    @functools.partial(
        pl.kernel,
        out_type=jax.ShapeDtypeStruct((_NC * _NS, rows_per_tile, feat),
                                      jnp.float32),
        mesh=mesh,
        scratch_types=[
            pltpu.VMEM((chunks_per_super, chunk), jnp.int32),
            pltpu.VMEM((chunks_per_super, chunk), jnp.int32),
            pltpu.VMEM((chunks_per_super, chunk), jnp.float32),
            pltpu.VMEM((chunk, feat), jnp.float32),
            pltpu.VMEM((zrows, feat), jnp.float32),
            pltpu.VMEM_SHARED((n_nodes, feat), jnp.float32),
            pltpu.SemaphoreType.DMA,
        ],
    )
    def body(x_hbm, alpha_hbm, idxi_hbm, idxj_hbm, out_hbm,
             idxj_v, idxi_v, alpha_v, g, zbuf, acc, sem):
        c = lax.axis_index("c")
        s = lax.axis_index("s")
        w = c * _NS + s

        @pl.loop(0, zrows)
        def _zrow(i):
            for t in range(feat // _LANES):
                zbuf[i, pl.ds(t * _LANES, _LANES)] = jnp.zeros(
                    (_LANES,), jnp.float32)

        base = s * rows_per_tile
        for z in range(n_z):
            pltpu.sync_copy(zbuf, acc.at[pl.ds(base + z * zrows, zrows)])
        plsc.subcore_barrier()

        @pl.loop(0, n_super)
        def _super(u):
            pltpu.sync_copy(idxj_hbm.at[w, u], idxj_v)
            pltpu.sync_copy(idxi_hbm.at[w, u], idxi_v)
            pltpu.sync_copy(alpha_hbm.at[w, u], alpha_v)

            @pl.loop(0, chunks_per_super)
            def _chunk(k):
                pltpu.async_copy(x_hbm.at[idxj_v.at[k]], g, sem).wait()

                @pl.loop(0, chunk // _LANES)
                def _sgrp(gg):
                    av = alpha_v[k, pl.ds(gg * _LANES, _LANES)]
                    for r16 in range(_LANES):
                        a = av[r16]
                        r = gg * _LANES + r16
                        for t in range(feat // _LANES):
                            sl = pl.ds(t * _LANES, _LANES)
                            g[r, sl] = g[r, sl] * a

                pltpu.sync_copy(g, acc.at[idxi_v.at[k]], add=True)

        plsc.subcore_barrier()
        pltpu.sync_copy(acc.at[pl.ds(base, rows_per_tile)], out_hbm.at[w])

    return body(x, alpha_r, idxi_r, idxj_r)


def _tc_finish(p0, p1, W, n_nodes, feat, block):
    def body(p0_ref, p1_ref, w_ref, o_ref):
        o_ref[...] = jnp.dot(p0_ref[...] + p1_ref[...], w_ref[...],
                             preferred_element_type=jnp.float32)

    return pl.pallas_call(
        body,
        grid=(n_nodes // block,),
        in_specs=[
            pl.BlockSpec((block, feat), lambda i: (i, 0)),
            pl.BlockSpec((block, feat), lambda i: (i, 0)),
            pl.BlockSpec((feat, feat), lambda i: (0, 0)),
        ],
        out_specs=pl.BlockSpec((block, feat), lambda i: (i, 0)),
        out_shape=jax.ShapeDtypeStruct((n_nodes, feat), jnp.float32),
    )(p0, p1, W)


def kernel(x, alpha_ij, idx_i, idx_j, W):
    n_nodes, feat = x.shape
    n_edges = alpha_ij.shape[0]
    nw = _NC * _NS
    chunk = 80
    n_super, cps = 5, 25
    assert nw * n_super * cps * chunk == n_edges

    shape = (nw, n_super, cps, chunk)
    idxi_r = idx_i.astype(jnp.int32).reshape(shape)
    idxj_r = idx_j.astype(jnp.int32).reshape(shape)
    alpha_r = alpha_ij.astype(jnp.float32).reshape(shape)

    partial = _sc_aggregate(x.astype(jnp.float32), alpha_r, idxi_r, idxj_r,
                            n_nodes, feat, n_super, cps, chunk)
    partial = partial.reshape(_NC, n_nodes, feat)
    return _tc_finish(partial[0], partial[1],
                      W.astype(jnp.float32), n_nodes, feat, 400)


# speedup vs baseline: 8.8072x; 1.4196x over previous
import functools

import jax
import jax.numpy as jnp
from jax import lax
from jax.experimental import pallas as pl
from jax.experimental.pallas import tpu as pltpu
from jax.experimental.pallas import tpu_sc as plsc

_NC = 2
_NS = 16
_LANES = 16


def _sc_aggregate(x, alpha_r, idxi_r, idxj_r, n_nodes, feat,
                  n_super, chunks_per_super, chunk):
    rows_per_tile = n_nodes // _NS
    n_z = 25
    zrows = rows_per_tile // n_z
    mesh = plsc.VectorSubcoreMesh(core_axis_name="c", subcore_axis_name="s")

    @functools.partial(
        pl.kernel,
        out_type=jax.ShapeDtypeStruct((_NC * _NS, rows_per_tile, feat),
                                      jnp.float32),
        mesh=mesh,
        scratch_types=[
            pltpu.VMEM((chunks_per_super, chunk), jnp.int32),
            pltpu.VMEM((chunks_per_super, chunk), jnp.int32),
            pltpu.VMEM((chunks_per_super, chunk), jnp.float32),
            pltpu.VMEM((chunk, feat), jnp.float32),
            pltpu.VMEM((chunk, feat), jnp.float32),
            pltpu.VMEM((zrows, feat), jnp.float32),
            pltpu.VMEM_SHARED((n_nodes, feat), jnp.float32),
            pltpu.SemaphoreType.DMA,
            pltpu.SemaphoreType.DMA,
        ],
    )
    def body(x_hbm, alpha_hbm, idxi_hbm, idxj_hbm, out_hbm,
             idxj_v, idxi_v, alpha_v, g0, g1, zbuf, acc, sem0, sem1):
        c = lax.axis_index("c")
        s = lax.axis_index("s")
        w = c * _NS + s

        @pl.loop(0, zrows)
        def _zrow(i):
            for t in range(feat // _LANES):
                zbuf[i, pl.ds(t * _LANES, _LANES)] = jnp.zeros(
                    (_LANES,), jnp.float32)

        base = s * rows_per_tile
        for z in range(n_z):
            pltpu.sync_copy(zbuf, acc.at[pl.ds(base + z * zrows, zrows)])
        plsc.subcore_barrier()

        gbufs = (g0, g1)
        sems = (sem0, sem1)

        @pl.loop(0, n_super)
        def _super(u):
            pltpu.sync_copy(idxj_hbm.at[w, u], idxj_v)
            pltpu.sync_copy(idxi_hbm.at[w, u], idxi_v)
            pltpu.sync_copy(alpha_hbm.at[w, u], alpha_v)

            pltpu.async_copy(x_hbm.at[idxj_v.at[0]], g0, sem0)

            def process_chunk(k, b, issue_next):
                g = gbufs[b]
                pltpu.make_async_copy(
                    x_hbm.at[idxj_v.at[k]], g, sems[b]).wait()

                if issue_next:
                    pltpu.async_copy(
                        x_hbm.at[idxj_v.at[k + 1]], gbufs[1 - b],
                        sems[1 - b])

                @pl.loop(0, chunk // _LANES)
                def _sgrp(gg):
                    av = alpha_v[k, pl.ds(gg * _LANES, _LANES)]
                    for r16 in range(_LANES):
                        a = av[r16]
                        r = gg * _LANES + r16
                        for t in range(feat // _LANES):
                            sl = pl.ds(t * _LANES, _LANES)
                            g[r, sl] = g[r, sl] * a

                pltpu.sync_copy(g, acc.at[idxi_v.at[k]], add=True)

            @pl.loop(0, chunks_per_super // 2)
            def _pair(kk):
                for b in range(2):
                    process_chunk(kk * 2 + b, b, issue_next=True)

            if chunks_per_super % 2:
                process_chunk(chunks_per_super - 1, 0, issue_next=False)

        plsc.subcore_barrier()
        pltpu.sync_copy(acc.at[pl.ds(base, rows_per_tile)], out_hbm.at[w])

    return body(x, alpha_r, idxi_r, idxj_r)


def _tc_finish(p0, p1, W, n_nodes, feat, block):
    def body(p0_ref, p1_ref, w_ref, o_ref):
        o_ref[...] = jnp.dot(p0_ref[...] + p1_ref[...], w_ref[...],
                             preferred_element_type=jnp.float32)

    return pl.pallas_call(
        body,
        grid=(n_nodes // block,),
        in_specs=[
            pl.BlockSpec((block, feat), lambda i: (i, 0)),
            pl.BlockSpec((block, feat), lambda i: (i, 0)),
            pl.BlockSpec((feat, feat), lambda i: (0, 0)),
        ],
        out_specs=pl.BlockSpec((block, feat), lambda i: (i, 0)),
        out_shape=jax.ShapeDtypeStruct((n_nodes, feat), jnp.float32),
    )(p0, p1, W)


def kernel(x, alpha_ij, idx_i, idx_j, W):
    n_nodes, feat = x.shape
    n_edges = alpha_ij.shape[0]
    nw = _NC * _NS
    chunk = 80
    n_super, cps = 5, 25
    assert nw * n_super * cps * chunk == n_edges

    shape = (nw, n_super, cps, chunk)
    idxi_r = idx_i.astype(jnp.int32).reshape(shape)
    idxj_r = idx_j.astype(jnp.int32).reshape(shape)
    alpha_r = alpha_ij.astype(jnp.float32).reshape(shape)

    partial = _sc_aggregate(x.astype(jnp.float32), alpha_r, idxi_r, idxj_r,
                            n_nodes, feat, n_super, cps, chunk)
    partial = partial.reshape(_NC, n_nodes, feat)
    return _tc_finish(partial[0], partial[1],
                      W.astype(jnp.float32), n_nodes, feat, 400)
